# Initial kernel scaffold; baseline (speedup 1.0000x reference)
import jax
import jax.numpy as jnp
from jax.experimental import pallas as pl


def kernel(champions, scalars, items, traits, champ_table, item_table, trait_table, bench_table, W1, b1, W2, b2):
    raise NotImplementedError("write your pallas kernel here")


# speedup vs baseline: 5.0559x; 5.0559x over previous
import jax
import jax.numpy as jnp
from jax import lax
from jax.experimental import pallas as pl

NC = 37
VEC = 142
NROW = 51
BB = 32


def _body(ch_ref, sc_ref, it_ref, tr_ref, ct_ref, itab_ref, ttab_ref, btab_ref,
          w1_ref, b1_ref, w2_ref, b2_ref, out_ref):
    f32 = jnp.float32
    ch = ch_ref[...]
    ids = ch[..., 1:11]
    stats = ch[..., 11:23]

    const = jnp.broadcast_to(ct_ref[0, :][None, None, :], (BB, NC, 30))
    parts = [const]
    for k in range(3):
        idk = ids[..., k][..., None]
        emb = jnp.where(idk == 0.0, itab_ref[0, :],
              jnp.where(idk == 1.0, itab_ref[1, :], itab_ref[2, :]))
        parts.append(emb)
    for k in range(7):
        idk = ids[..., 3 + k][..., None]
        acc = jnp.broadcast_to(ttab_ref[6, :], (BB, NC, 10))
        for r in range(5, -1, -1):
            acc = jnp.where(idk == float(r), ttab_ref[r, :], acc)
        parts.append(acc)
    parts.append(stats)
    out_ref[:, 0:NC, :] = jnp.concatenate(parts, axis=-1)

    x = jnp.clip(sc_ref[...], 0.0, 200.0) * ((VEC - 1) / 200.0)
    low = jnp.floor(x)
    frac = (x - low)[..., None]
    lowb = low[..., None]
    high = jnp.minimum(lowb + 1.0, float(VEC - 1))
    p = lax.broadcasted_iota(jnp.int32, (BB, 3, VEC), 2).astype(f32)
    enc = jnp.where(p == lowb, 1.0 - frac, 0.0) + jnp.where(p == high, frac, 0.0)
    out_ref[:, NC:NC + 3, :] = enc

    bid = it_ref[...][..., None]
    acc = jnp.broadcast_to(btab_ref[9, :], (BB, 10, VEC))
    for r in range(8, -1, -1):
        acc = jnp.where(bid == r, btab_ref[r, :], acc)
    out_ref[:, NC + 3:NC + 13, :] = acc

    h = jnp.maximum(
        jnp.dot(tr_ref[...], w1_ref[...], preferred_element_type=f32) + b1_ref[0, :], 0.0)
    y = jnp.dot(h, w2_ref[...], preferred_element_type=f32) + b2_ref[0, :]
    out_ref[:, NC + 13:NROW, :] = y[:, None, :]


def kernel(champions, scalars, items, traits, champ_table, item_table, trait_table,
           bench_table, W1, b1, W2, b2):
    B = champions.shape[0]
    full = lambda shp: pl.BlockSpec(shp, lambda i: (0,) * len(shp))
    return pl.pallas_call(
        _body,
        grid=(B // BB,),
        in_specs=[
            pl.BlockSpec((BB, NC, 23), lambda i: (i, 0, 0)),
            pl.BlockSpec((BB, 3), lambda i: (i, 0)),
            pl.BlockSpec((BB, 10), lambda i: (i, 0)),
            pl.BlockSpec((BB, 26), lambda i: (i, 0)),
            full((1, 30)), full((3, 10)), full((7, 10)), full((10, VEC)),
            full((26, 26)), full((1, 26)), full((26, VEC)), full((1, VEC)),
        ],
        out_specs=pl.BlockSpec((BB, NROW, VEC), lambda i: (i, 0, 0)),
        out_shape=jax.ShapeDtypeStruct((B, NROW, VEC), jnp.float32),
    )(champions, scalars, items, traits, champ_table, item_table, trait_table,
      bench_table, W1, b1.reshape(1, 26), W2, b2.reshape(1, VEC))


# speedup vs baseline: 17.4680x; 3.4550x over previous
import numpy as np
import jax
import jax.numpy as jnp
from jax import lax
from jax.experimental import pallas as pl

NC = 37
VEC = 142
NROW = 51
NF = 71
BB = 64

_S = np.zeros((23, NF), np.float32)
_R = np.zeros((NF,), np.float32)
_MSK = np.zeros((NF,), np.float32)
for _k in range(3):
    for _r in range(3):
        _j = 1 + 3 * _k + _r
        _S[1 + _k, _j] = 1.0
        _R[_j] = _r
        _MSK[_j] = 1.0
for _k in range(7):
    for _r in range(7):
        _j = 10 + 7 * _k + _r
        _S[4 + _k, _j] = 1.0
        _R[_j] = _r
        _MSK[_j] = 1.0
for _j in range(12):
    _S[11 + _j, 59 + _j] = 1.0
_E0 = np.zeros((NF,), np.float32)
_E0[0] = 1.0


def _body(ch_ref, sc_ref, it_ref, tr_ref, s_ref, aux_ref, m_ref, w1_ref, b1_ref,
          w2_ref, b2_ref, btab_ref, out_ref):
    f32 = jnp.float32
    ch2 = ch_ref[...]
    G = jnp.dot(ch2, s_ref[...], preferred_element_type=f32)
    msk = aux_ref[1, :][None, :]
    F = (msk * (G == aux_ref[0, :][None, :]).astype(f32)
         + (1.0 - msk) * G + aux_ref[2, :][None, :])
    rows = jnp.dot(F, m_ref[...], preferred_element_type=f32)
    out_ref[:, 0:NC, :] = rows.reshape(BB, NC, VEC)

    x = jnp.clip(sc_ref[...], 0.0, 200.0) * ((VEC - 1) / 200.0)
    low = jnp.floor(x)
    frac = (x - low)[..., None]
    lowb = low[..., None]
    high = jnp.minimum(lowb + 1.0, float(VEC - 1))
    p = lax.broadcasted_iota(jnp.int32, (BB, 3, VEC), 2).astype(f32)
    enc = jnp.where(p == lowb, 1.0 - frac, 0.0) + jnp.where(p == high, frac, 0.0)
    out_ref[:, NC:NC + 3, :] = enc

    it2 = it_ref[...]
    oh = (lax.broadcasted_iota(jnp.int32, (BB * 10, 10), 1) == it2).astype(f32)
    bench = jnp.dot(oh, btab_ref[...], preferred_element_type=f32)
    out_ref[:, NC + 3:NC + 13, :] = bench.reshape(BB, 10, VEC)

    h = jnp.maximum(
        jnp.dot(tr_ref[...], w1_ref[...], preferred_element_type=f32) + b1_ref[0, :], 0.0)
    y = jnp.dot(h, w2_ref[...], preferred_element_type=f32) + b2_ref[0, :]
    out_ref[:, NC + 13:NROW, :] = y[:, None, :]


def kernel(champions, scalars, items, traits, champ_table, item_table, trait_table,
           bench_table, W1, b1, W2, b2):
    B = champions.shape[0]
    f32 = jnp.float32
    M = jnp.zeros((NF, VEC), f32)
    M = M.at[0, 0:30].set(champ_table[0])
    for k in range(3):
        M = M.at[1 + 3 * k:4 + 3 * k, 30 + 10 * k:40 + 10 * k].set(item_table)
    for k in range(7):
        M = M.at[10 + 7 * k:17 + 7 * k, 60 + 10 * k:70 + 10 * k].set(trait_table)
    M = M.at[59:NF, 130:VEC].set(jnp.eye(12, dtype=f32))

    full = lambda shp: pl.BlockSpec(shp, lambda i: (0,) * len(shp))
    return pl.pallas_call(
        _body,
        grid=(B // BB,),
        in_specs=[
            pl.BlockSpec((BB * NC, 23), lambda i: (i, 0)),
            pl.BlockSpec((BB, 3), lambda i: (i, 0)),
            pl.BlockSpec((BB * 10, 1), lambda i: (i, 0)),
            pl.BlockSpec((BB, 26), lambda i: (i, 0)),
            full((23, NF)), full((3, NF)), full((NF, VEC)),
            full((26, 26)), full((1, 26)), full((26, VEC)), full((1, VEC)),
            full((10, VEC)),
        ],
        out_specs=pl.BlockSpec((BB, NROW, VEC), lambda i: (i, 0, 0)),
        out_shape=jax.ShapeDtypeStruct((B, NROW, VEC), jnp.float32),
    )(champions.reshape(B * NC, 23), scalars, items.reshape(B * 10, 1), traits,
      jnp.asarray(_S), jnp.asarray(np.stack([_R, _MSK, _E0])), M,
      W1, b1.reshape(1, 26), W2, b2.reshape(1, VEC), bench_table)


# speedup vs baseline: 20.8755x; 1.1951x over previous
import numpy as np
import jax
import jax.numpy as jnp
from jax import lax
from jax.experimental import pallas as pl

NC = 37
NCP = 40
VEC = 142
NROW = 51
NF = 71
BB = 64

_S = np.zeros((23, NF), np.float32)
_R = np.zeros((NF,), np.float32)
_MSK = np.zeros((NF,), np.float32)
for _k in range(3):
    for _r in range(3):
        _j = 1 + 3 * _k + _r
        _S[1 + _k, _j] = 1.0
        _R[_j] = _r
        _MSK[_j] = 1.0
for _k in range(7):
    for _r in range(7):
        _j = 10 + 7 * _k + _r
        _S[4 + _k, _j] = 1.0
        _R[_j] = _r
        _MSK[_j] = 1.0
for _j in range(12):
    _S[11 + _j, 59 + _j] = 1.0
_E0 = np.zeros((NF,), np.float32)
_E0[0] = 1.0


def _body(ch_ref, sc_ref, it_ref, tr_ref, s_ref, aux_ref, m_ref, w1_ref, b1_ref,
          w2_ref, b2_ref, btab_ref, out_ref):
    f32 = jnp.float32
    ch2 = ch_ref[...]
    G = jnp.dot(ch2, s_ref[...], preferred_element_type=f32)
    msk = aux_ref[1, :][None, :] != 0.0
    F = jnp.where(msk, (G == aux_ref[0, :][None, :]).astype(f32), G) + aux_ref[2, :][None, :]
    rows = jnp.dot(F, m_ref[...], preferred_element_type=f32)
    out_ref[:, 0:NC, :] = rows.reshape(BB, NCP, VEC)[:, 0:NC, :]

    x = jnp.clip(sc_ref[...], 0.0, 200.0) * ((VEC - 1) / 200.0)
    low = jnp.floor(x)
    frac = (x - low)[..., None]
    lowb = low[..., None]
    high = jnp.minimum(lowb + 1.0, float(VEC - 1))
    p = lax.broadcasted_iota(jnp.int32, (BB, 3, VEC), 2).astype(f32)
    enc = jnp.where(p == lowb, 1.0 - frac, 0.0) + jnp.where(p == high, frac, 0.0)
    out_ref[:, NC:NC + 3, :] = enc

    it2 = it_ref[...]
    oh = (lax.broadcasted_iota(jnp.int32, (BB * 16, 10), 1) == it2).astype(f32)
    bench = jnp.dot(oh, btab_ref[...], preferred_element_type=f32)
    out_ref[:, NC + 3:NC + 13, :] = bench.reshape(BB, 16, VEC)[:, 0:10, :]

    h = jnp.maximum(
        jnp.dot(tr_ref[...], w1_ref[...], preferred_element_type=f32) + b1_ref[0, :], 0.0)
    y = jnp.dot(h, w2_ref[...], preferred_element_type=f32) + b2_ref[0, :]
    out_ref[:, NC + 13:NROW, :] = y[:, None, :]


def kernel(champions, scalars, items, traits, champ_table, item_table, trait_table,
           bench_table, W1, b1, W2, b2):
    B = champions.shape[0]
    f32 = jnp.float32
    M = jnp.zeros((NF, VEC), f32)
    M = M.at[0, 0:30].set(champ_table[0])
    for k in range(3):
        M = M.at[1 + 3 * k:4 + 3 * k, 30 + 10 * k:40 + 10 * k].set(item_table)
    for k in range(7):
        M = M.at[10 + 7 * k:17 + 7 * k, 60 + 10 * k:70 + 10 * k].set(trait_table)
    M = M.at[59:NF, 130:VEC].set(jnp.eye(12, dtype=f32))

    ch40 = jnp.pad(champions, ((0, 0), (0, NCP - NC), (0, 0))).reshape(B * NCP, 23)
    it16 = jnp.pad(items, ((0, 0), (0, 6))).reshape(B * 16, 1)

    full = lambda shp: pl.BlockSpec(shp, lambda i: (0,) * len(shp))
    return pl.pallas_call(
        _body,
        grid=(B // BB,),
        in_specs=[
            pl.BlockSpec((BB * NCP, 23), lambda i: (i, 0)),
            pl.BlockSpec((BB, 3), lambda i: (i, 0)),
            pl.BlockSpec((BB * 16, 1), lambda i: (i, 0)),
            pl.BlockSpec((BB, 26), lambda i: (i, 0)),
            full((23, NF)), full((3, NF)), full((NF, VEC)),
            full((26, 26)), full((1, 26)), full((26, VEC)), full((1, VEC)),
            full((10, VEC)),
        ],
        out_specs=pl.BlockSpec((BB, NROW, VEC), lambda i: (i, 0, 0)),
        out_shape=jax.ShapeDtypeStruct((B, NROW, VEC), jnp.float32),
    )(ch40, scalars, it16, traits,
      jnp.asarray(_S), jnp.asarray(np.stack([_R, _MSK, _E0])), M,
      W1, b1.reshape(1, 26), W2, b2.reshape(1, VEC), bench_table)


# speedup vs baseline: 24.0104x; 1.1502x over previous
import numpy as np
import jax
import jax.numpy as jnp
from jax import lax
from jax.experimental import pallas as pl

NC = 37
NCP = 40
VEC = 142
NROW = 51
NF = 71
BB = 64

_S = np.zeros((23, NF), np.float32)
_R = np.zeros((NF,), np.float32)
_MSK = np.zeros((NF,), np.float32)
for _k in range(3):
    for _r in range(3):
        _j = 1 + 3 * _k + _r
        _S[1 + _k, _j] = 1.0
        _R[_j] = _r
        _MSK[_j] = 1.0
for _k in range(7):
    for _r in range(7):
        _j = 10 + 7 * _k + _r
        _S[4 + _k, _j] = 1.0
        _R[_j] = _r
        _MSK[_j] = 1.0
for _j in range(12):
    _S[11 + _j, 59 + _j] = 1.0
_E0 = np.zeros((NF,), np.float32)
_E0[0] = 1.0


def _body(ch_ref, sc_ref, it_ref, tr_ref, s_ref, aux_ref, m_ref, w1_ref, b1_ref,
          w2_ref, b2_ref, btab_ref, out_ref):
    f32 = jnp.float32
    ch2 = ch_ref[...]
    G = jnp.dot(ch2, s_ref[...], preferred_element_type=f32)
    msk = aux_ref[1, :][None, :] != 0.0
    F = jnp.where(msk, (G == aux_ref[0, :][None, :]).astype(f32), G) + aux_ref[2, :][None, :]
    rows = jnp.dot(F, m_ref[...], preferred_element_type=f32)
    out_ref[:, 0:NC, 0:VEC] = rows.reshape(BB, NCP, VEC)[:, 0:NC, :]

    x = jnp.clip(sc_ref[...], 0.0, 200.0) * ((VEC - 1) / 200.0)
    low = jnp.floor(x)
    frac = (x - low)[..., None]
    lowb = low[..., None]
    high = jnp.minimum(lowb + 1.0, float(VEC - 1))
    p = lax.broadcasted_iota(jnp.int32, (BB, 3, VEC), 2).astype(f32)
    enc = jnp.where(p == lowb, 1.0 - frac, 0.0) + jnp.where(p == high, frac, 0.0)
    out_ref[:, NC:NC + 3, 0:VEC] = enc

    it2 = it_ref[...]
    oh = (lax.broadcasted_iota(jnp.int32, (BB * 16, 10), 1) == it2).astype(f32)
    bench = jnp.dot(oh, btab_ref[...], preferred_element_type=f32)
    out_ref[:, NC + 3:NC + 13, 0:VEC] = bench.reshape(BB, 16, VEC)[:, 0:10, :]

    h = jnp.maximum(
        jnp.dot(tr_ref[...], w1_ref[...], preferred_element_type=f32) + b1_ref[0, :], 0.0)
    y = jnp.dot(h, w2_ref[...], preferred_element_type=f32) + b2_ref[0, :]
    out_ref[:, NC + 13:NROW, 0:VEC] = y[:, None, :]


def kernel(champions, scalars, items, traits, champ_table, item_table, trait_table,
           bench_table, W1, b1, W2, b2):
    B = champions.shape[0]
    f32 = jnp.float32
    M = jnp.zeros((NF, VEC), f32)
    M = M.at[0, 0:30].set(champ_table[0])
    for k in range(3):
        M = M.at[1 + 3 * k:4 + 3 * k, 30 + 10 * k:40 + 10 * k].set(item_table)
    for k in range(7):
        M = M.at[10 + 7 * k:17 + 7 * k, 60 + 10 * k:70 + 10 * k].set(trait_table)
    M = M.at[59:NF, 130:VEC].set(jnp.eye(12, dtype=f32))

    ch40 = jnp.pad(champions, ((0, 0), (0, NCP - NC), (0, 0))).reshape(B * NCP, 23)
    it16 = jnp.pad(items, ((0, 0), (0, 6))).reshape(B * 16, 1)

    full = lambda shp: pl.BlockSpec(shp, lambda i: (0,) * len(shp))
    padded = pl.pallas_call(
        _body,
        grid=(B // BB,),
        in_specs=[
            pl.BlockSpec((BB * NCP, 23), lambda i: (i, 0)),
            pl.BlockSpec((BB, 3), lambda i: (i, 0)),
            pl.BlockSpec((BB * 16, 1), lambda i: (i, 0)),
            pl.BlockSpec((BB, 26), lambda i: (i, 0)),
            full((23, NF)), full((3, NF)), full((NF, VEC)),
            full((26, 26)), full((1, 26)), full((26, VEC)), full((1, VEC)),
            full((10, VEC)),
        ],
        out_specs=pl.BlockSpec((BB, 56, 256), lambda i: (i, 0, 0)),
        out_shape=jax.ShapeDtypeStruct((B, 56, 256), jnp.float32),
    )(ch40, scalars, it16, traits,
      jnp.asarray(_S), jnp.asarray(np.stack([_R, _MSK, _E0])), M,
      W1, b1.reshape(1, 26), W2, b2.reshape(1, VEC), bench_table)
    return padded[:, 0:NROW, 0:VEC]
